# baseline (speedup 1.0000x reference)
import math

import numpy as np

import jax
import jax.numpy as jnp
from jax.experimental import pallas as pl
from jax.experimental.pallas import tpu as pltpu

_EPS = 1e-10
_FILT = 0.7
_TINY = np.float32(np.finfo(np.float32).tiny)


def _diag_mask(n):
    r = jax.lax.broadcasted_iota(jnp.int32, (n, n), 0)
    c = jax.lax.broadcasted_iota(jnp.int32, (n, n), 1)
    return r == c


def _attention(adj, h, m, wb):
    n = adj.shape[0]
    att = jnp.dot(h, wb, preferred_element_type=jnp.float32)
    att = att + (m - 1.0) * 1e10
    e = jnp.exp(att - jnp.max(att, axis=0, keepdims=True))
    denom = jnp.dot(adj, e, preferred_element_type=jnp.float32) + _EPS
    dm = _diag_mask(n)
    diag_a = jnp.sum(jnp.where(dm, adj, 0.0), axis=1, keepdims=True)
    rowsum = jnp.sum(adj, axis=1, keepdims=True)
    return e * diag_a / denom * rowsum * m


def _lane_dense(v, n):
    return jnp.sum(jnp.where(_diag_mask(n), v, 0.0), axis=0, keepdims=True)


def _threefry_rounds(x0, x1, k0, k1):
    ks2 = k0 ^ k1 ^ np.uint32(0x1BD11BDA)
    ks = (k0, k1, ks2)
    rots = ((13, 15, 26, 6), (17, 29, 16, 24))
    x0 = x0 + k0
    x1 = x1 + k1
    for i in range(5):
        for r in rots[i % 2]:
            x0 = x0 + x1
            x1 = (x1 << np.uint32(r)) | (x1 >> np.uint32(32 - r))
            x1 = x0 ^ x1
        x0 = x0 + ks[(i + 1) % 3]
        x1 = x1 + ks[(i + 2) % 3] + np.uint32(i + 1)
    return x0, x1


def _gumbel_argmax_rows(k0, k1, logits_row, base, nrow, ncat):
    r = jax.lax.broadcasted_iota(jnp.uint32, (nrow, ncat), 0)
    c = jax.lax.broadcasted_iota(jnp.uint32, (nrow, ncat), 1)
    f = r * np.uint32(ncat) + c + np.uint32(base)
    y0, y1 = _threefry_rounds(jnp.zeros((nrow, ncat), jnp.uint32), f, k0, k1)
    bits = y0 ^ y1
    ub = (bits >> np.uint32(9)) | np.uint32(0x3F800000)
    u = jax.lax.bitcast_convert_type(ub, jnp.float32) - 1.0
    u = jnp.maximum(_TINY, u + _TINY)
    g = -jnp.log(-jnp.log(u))
    x = g + logits_row
    rowmax = jnp.max(x, axis=1, keepdims=True)
    ci = jax.lax.broadcasted_iota(jnp.int32, (nrow, ncat), 1)
    return jnp.min(jnp.where(x == rowmax, ci, ncat), axis=1, keepdims=True)


def _sample_to_ref(k0_ref, k1_ref, att_b, m, nsamp, chunk, idx_ref):
    n = att_b.shape[0]
    att_row = _lane_dense(att_b, n)
    m_row = _lane_dense(m, n)
    p = att_row * m_row
    p = p / (jnp.sum(p, axis=1, keepdims=True) + _EPS)
    logits = jnp.log(p + 1e-30)
    k0 = k0_ref[0][0:1, 0:1]
    k1 = k1_ref[0][0:1, 0:1]
    for r0 in range(0, nsamp, chunk):
        rows = min(chunk, nsamp - r0)
        idx = _gumbel_argmax_rows(k0, k1, logits, r0 * n, rows, n)
        idx_ref[0, pl.ds(r0, rows), :] = idx


def _block0_body(x_ref, m_ref, k0_ref, k1_ref, w0_ref, b0_ref, w1_ref,
                 b1_ref, w2_ref, b2_ref, wb_ref, adj_ref, idx_ref, z_ref):
    x = x_ref[0]
    m = m_ref[0]

    nrm = jnp.sqrt(jnp.sum(x * x, axis=-1, keepdims=True))
    xn = x / jnp.maximum(nrm, 1e-12)
    a = jax.lax.dot_general(xn, xn, (((1,), (1,)), ((), ())),
                            preferred_element_type=jnp.float32)
    a = 0.5 * jnp.tanh(a) + 0.5
    deg_c = jnp.sum(a, axis=1, keepdims=True)
    deg_c = jnp.where(deg_c == 0.0, 1e-10, deg_c)
    deg_r = jnp.sum(a, axis=0, keepdims=True)
    deg_r = jnp.where(deg_r == 0.0, 1e-10, deg_r)
    adj = jax.lax.rsqrt(deg_c) * a * jax.lax.rsqrt(deg_r)
    adj_ref[0] = adj

    h = xn
    for w, b in ((w0_ref, b0_ref), (w1_ref, b1_ref), (w2_ref, b2_ref)):
        y = jnp.dot(adj, h, preferred_element_type=jnp.float32)
        y = jnp.dot(y, w[...], preferred_element_type=jnp.float32) + b[...]
        h = jnp.maximum(y, 0.0)
    h = m * h

    att_b = _attention(adj, h, m, wb_ref[...])
    z_ref[0] = att_b * h
    nsamp = idx_ref.shape[1]
    _sample_to_ref(k0_ref, k1_ref, att_b, m, nsamp, 72, idx_ref)


def _pool_gcn_body(idx_ref, val_ref, k0_ref, k1_ref, z_ref, adj_ref, w_ref,
                   b_ref, wb_ref, nadj_ref, idx1_ref, z1_ref):
    idx = idx_ref[0]
    val = val_ref[0]
    z = z_ref[0]
    a = adj_ref[0]
    k = idx.shape[0]
    n = a.shape[0]

    cols = jax.lax.broadcasted_iota(jnp.int32, (k, n), 1)
    sel = jnp.where(cols == idx, val, 0.0)
    assign = jnp.dot(sel, a, preferred_element_type=jnp.float32)
    colsum = jnp.sum(assign, axis=0, keepdims=True)
    sn = assign / (colsum + _EPS)
    feat = jnp.dot(sn, z, preferred_element_type=jnp.float32)
    tmp = jnp.dot(sn, a, preferred_element_type=jnp.float32)
    nadj = jax.lax.dot_general(tmp, sn, (((1,), (1,)), ((), ())),
                               preferred_element_type=jnp.float32)
    nadj_ref[0] = nadj

    y = jnp.dot(nadj, feat, preferred_element_type=jnp.float32)
    y = jnp.dot(y, w_ref[...], preferred_element_type=jnp.float32) + b_ref[...]
    h = jnp.maximum(y, 0.0)
    h = val * h

    att_b = _attention(nadj, h, val, wb_ref[...])
    z1_ref[0] = att_b * h
    nsamp = idx1_ref.shape[1]
    _sample_to_ref(k0_ref, k1_ref, att_b, val, nsamp, 84, idx1_ref)


def _pool_final_body(idx_ref, val_ref, z_ref, adj_ref, h_ref):
    idx = idx_ref[0]
    val = val_ref[0]
    z = z_ref[0]
    a = adj_ref[0]
    k = idx.shape[0]
    n = a.shape[0]

    cols = jax.lax.broadcasted_iota(jnp.int32, (k, n), 1)
    sel = jnp.where(cols == idx, val, 0.0)
    assign = jnp.dot(sel, a, preferred_element_type=jnp.float32)
    colsum = jnp.sum(assign, axis=0, keepdims=True)
    sn = assign / (colsum + _EPS)
    h_ref[0] = jnp.dot(sn, z, preferred_element_type=jnp.float32)


def _whole(shape):
    nd = len(shape)
    return pl.BlockSpec((1,) + shape[1:], lambda b: (b,) + (0,) * (nd - 1))


def _bcast(arr):
    return pl.BlockSpec(arr.shape, lambda b: (0,) * arr.ndim)


_PAR = pltpu.CompilerParams(dimension_semantics=("parallel",))


def _key_lanes(key, bsz):
    kd = jax.random.key_data(jax.random.split(key, bsz))
    k0 = jnp.broadcast_to(kd[:, 0:1], (bsz, 128)).reshape(bsz, 1, 128)
    k1 = jnp.broadcast_to(kd[:, 1:2], (bsz, 128)).reshape(bsz, 1, 128)
    return k0, k1


def _valid_rows(mask, k_max):
    bsz = mask.shape[0]
    k_list = jnp.ceil(_FILT * jnp.sum(mask, axis=1)).astype(jnp.int32)
    return (jax.lax.broadcasted_iota(jnp.int32, (bsz, k_max), 1)
            < k_list[:, None]).astype(jnp.float32)


def kernel(node_feat, mask_node, g0_w, g0_b, g1_w, g1_b, g2_w, g2_b,
           wb0, g3_w, g3_b, wb1, samp_key):
    bsz, n, _ = node_feat.shape
    hid = g0_w.shape[1]
    k0n = int(math.ceil(_FILT * n))
    k1n = int(math.ceil(_FILT * k0n))

    key = jax.random.key(samp_key)
    keys = jax.random.split(key, 2)
    ka0, ka1 = _key_lanes(keys[0], bsz)
    kb0, kb1 = _key_lanes(keys[1], bsz)

    adj, idx0, z0 = pl.pallas_call(
        _block0_body,
        out_shape=(jax.ShapeDtypeStruct((bsz, n, n), jnp.float32),
                   jax.ShapeDtypeStruct((bsz, k0n, 1), jnp.int32),
                   jax.ShapeDtypeStruct((bsz, n, hid), jnp.float32)),
        grid=(bsz,),
        in_specs=[_whole((bsz, n, node_feat.shape[2])),
                  _whole((bsz, n, 1)),
                  _whole((bsz, 1, 128)), _whole((bsz, 1, 128)),
                  _bcast(g0_w), _bcast(g0_b), _bcast(g1_w), _bcast(g1_b),
                  _bcast(g2_w), _bcast(g2_b), _bcast(wb0)],
        out_specs=(_whole((bsz, n, n)), _whole((bsz, k0n, 1)),
                   _whole((bsz, n, hid))),
        compiler_params=_PAR,
    )(node_feat, mask_node.reshape(bsz, n, 1), ka0, ka1, g0_w, g0_b,
      g1_w, g1_b, g2_w, g2_b, wb0)

    mask1 = _valid_rows(mask_node, k0n)

    nadj, idx1, z1 = pl.pallas_call(
        _pool_gcn_body,
        out_shape=(jax.ShapeDtypeStruct((bsz, k0n, k0n), jnp.float32),
                   jax.ShapeDtypeStruct((bsz, k1n, 1), jnp.int32),
                   jax.ShapeDtypeStruct((bsz, k0n, hid), jnp.float32)),
        grid=(bsz,),
        in_specs=[_whole((bsz, k0n, 1)), _whole((bsz, k0n, 1)),
                  _whole((bsz, 1, 128)), _whole((bsz, 1, 128)),
                  _whole((bsz, n, hid)), _whole((bsz, n, n)),
                  _bcast(g3_w), _bcast(g3_b), _bcast(wb1)],
        out_specs=(_whole((bsz, k0n, k0n)), _whole((bsz, k1n, 1)),
                   _whole((bsz, k0n, hid))),
        compiler_params=_PAR,
    )(idx0, mask1.reshape(bsz, k0n, 1), kb0, kb1, z0, adj, g3_w, g3_b, wb1)

    mask2 = _valid_rows(mask1, k1n)

    x_out = pl.pallas_call(
        _pool_final_body,
        out_shape=jax.ShapeDtypeStruct((bsz, k1n, hid), jnp.float32),
        grid=(bsz,),
        in_specs=[_whole((bsz, k1n, 1)), _whole((bsz, k1n, 1)),
                  _whole((bsz, k0n, hid)), _whole((bsz, k0n, k0n))],
        out_specs=_whole((bsz, k1n, hid)),
        compiler_params=_PAR,
    )(idx1, mask2.reshape(bsz, k1n, 1), z1, nadj)

    return x_out


# speedup vs baseline: 1.0939x; 1.0939x over previous
import math

import numpy as np

import jax
import jax.numpy as jnp
from jax.experimental import pallas as pl
from jax.experimental.pallas import tpu as pltpu

_EPS = 1e-10
_FILT = 0.7
_TINY = np.float32(np.finfo(np.float32).tiny)


def _diag_mask(n):
    r = jax.lax.broadcasted_iota(jnp.int32, (n, n), 0)
    c = jax.lax.broadcasted_iota(jnp.int32, (n, n), 1)
    return r == c


def _attention(adj, h, m, wb):
    n = adj.shape[0]
    att = jnp.dot(h, wb, preferred_element_type=jnp.float32)
    att = att + (m - 1.0) * 1e10
    e = jnp.exp(att - jnp.max(att, axis=0, keepdims=True))
    denom = jnp.dot(adj, e, preferred_element_type=jnp.float32) + _EPS
    dm = _diag_mask(n)
    diag_a = jnp.sum(jnp.where(dm, adj, 0.0), axis=1, keepdims=True)
    rowsum = jnp.sum(adj, axis=1, keepdims=True)
    return e * diag_a / denom * rowsum * m


def _lane_dense(v, n):
    return jnp.sum(jnp.where(_diag_mask(n), v, 0.0), axis=0, keepdims=True)


def _threefry_rounds(x0, x1, k0, k1):
    ks2 = k0 ^ k1 ^ np.uint32(0x1BD11BDA)
    ks = (k0, k1, ks2)
    rots = ((13, 15, 26, 6), (17, 29, 16, 24))
    for i in range(5):
        for r in rots[i % 2]:
            x0 = x0 + x1
            x1 = (x1 << np.uint32(r)) | (x1 >> np.uint32(32 - r))
            x1 = x0 ^ x1
        x0 = x0 + ks[(i + 1) % 3]
        x1 = x1 + ks[(i + 2) % 3] + np.uint32(i + 1)
    return x0, x1


def _gumbel_argmax_rows(k0, k1, logits_row, base, nrow, ncat):
    rowoff = (jax.lax.broadcasted_iota(jnp.uint32, (nrow, 1), 0)
              * np.uint32(ncat) + np.uint32(base)) + k1
    c = jax.lax.broadcasted_iota(jnp.uint32, (nrow, ncat), 1)
    x1 = c + rowoff
    x0 = jnp.broadcast_to(k0, (nrow, ncat))
    y0, y1 = _threefry_rounds(x0, x1, k0, k1)
    bits = y0 ^ y1
    ub = (bits >> np.uint32(9)) | np.uint32(0x3F800000)
    u = jax.lax.bitcast_convert_type(ub, jnp.float32) - 1.0
    u = jnp.maximum(_TINY, u + _TINY)
    g = -jnp.log(-jnp.log(u))
    x = g + logits_row
    rowmax = jnp.max(x, axis=1, keepdims=True)
    ci = jax.lax.broadcasted_iota(jnp.int32, (nrow, ncat), 1)
    return jnp.min(jnp.where(x == rowmax, ci, ncat), axis=1, keepdims=True)


def _sample_to_ref(k0_ref, k1_ref, att_b, m, nsamp, chunk, idx_ref):
    n = att_b.shape[0]
    att_row = _lane_dense(att_b, n)
    m_row = _lane_dense(m, n)
    p = att_row * m_row
    p = p / (jnp.sum(p, axis=1, keepdims=True) + _EPS)
    logits = jnp.log(p + 1e-30)
    k0 = k0_ref[0][0:1, 0:1]
    k1 = k1_ref[0][0:1, 0:1]
    for r0 in range(0, nsamp, chunk):
        rows = min(chunk, nsamp - r0)
        idx = _gumbel_argmax_rows(k0, k1, logits, r0 * n, rows, n)
        idx_ref[0, pl.ds(r0, rows), :] = idx


def _block0_body(x_ref, m_ref, k0_ref, k1_ref, w0_ref, b0_ref, w1_ref,
                 b1_ref, w2_ref, b2_ref, wb_ref, adj_ref, idx_ref, z_ref):
    x = x_ref[0]
    m = m_ref[0]

    nrm = jnp.sqrt(jnp.sum(x * x, axis=-1, keepdims=True))
    xn = x / jnp.maximum(nrm, 1e-12)
    a = jax.lax.dot_general(xn, xn, (((1,), (1,)), ((), ())),
                            preferred_element_type=jnp.float32)
    a = 0.5 * jnp.tanh(a) + 0.5
    deg_c = jnp.sum(a, axis=1, keepdims=True)
    deg_c = jnp.where(deg_c == 0.0, 1e-10, deg_c)
    deg_r = jnp.sum(a, axis=0, keepdims=True)
    deg_r = jnp.where(deg_r == 0.0, 1e-10, deg_r)
    adj = jax.lax.rsqrt(deg_c) * a * jax.lax.rsqrt(deg_r)
    adj_ref[0] = adj

    h = xn
    for w, b in ((w0_ref, b0_ref), (w1_ref, b1_ref), (w2_ref, b2_ref)):
        y = jnp.dot(adj, h, preferred_element_type=jnp.float32)
        y = jnp.dot(y, w[...], preferred_element_type=jnp.float32) + b[...]
        h = jnp.maximum(y, 0.0)
    h = m * h

    att_b = _attention(adj, h, m, wb_ref[...])
    z_ref[0] = att_b * h
    nsamp = idx_ref.shape[1]
    _sample_to_ref(k0_ref, k1_ref, att_b, m, nsamp, 24, idx_ref)


def _pool_gcn_body(idx_ref, val_ref, k0_ref, k1_ref, z_ref, adj_ref, w_ref,
                   b_ref, wb_ref, nadj_ref, idx1_ref, z1_ref):
    idx = idx_ref[0]
    val = val_ref[0]
    z = z_ref[0]
    a = adj_ref[0]
    k = idx.shape[0]
    n = a.shape[0]

    cols = jax.lax.broadcasted_iota(jnp.int32, (k, n), 1)
    sel = jnp.where(cols == idx, val, 0.0)
    assign = jnp.dot(sel, a, preferred_element_type=jnp.float32)
    colsum = jnp.sum(assign, axis=0, keepdims=True)
    sn = assign / (colsum + _EPS)
    feat = jnp.dot(sn, z, preferred_element_type=jnp.float32)
    tmp = jnp.dot(sn, a, preferred_element_type=jnp.float32)
    nadj = jax.lax.dot_general(tmp, sn, (((1,), (1,)), ((), ())),
                               preferred_element_type=jnp.float32)
    nadj_ref[0] = nadj

    y = jnp.dot(nadj, feat, preferred_element_type=jnp.float32)
    y = jnp.dot(y, w_ref[...], preferred_element_type=jnp.float32) + b_ref[...]
    h = jnp.maximum(y, 0.0)
    h = val * h

    att_b = _attention(nadj, h, val, wb_ref[...])
    z1_ref[0] = att_b * h
    nsamp = idx1_ref.shape[1]
    _sample_to_ref(k0_ref, k1_ref, att_b, val, nsamp, 28, idx1_ref)


def _pool_final_body(idx_ref, val_ref, z_ref, adj_ref, h_ref):
    idx = idx_ref[0]
    val = val_ref[0]
    z = z_ref[0]
    a = adj_ref[0]
    k = idx.shape[0]
    n = a.shape[0]

    cols = jax.lax.broadcasted_iota(jnp.int32, (k, n), 1)
    sel = jnp.where(cols == idx, val, 0.0)
    assign = jnp.dot(sel, a, preferred_element_type=jnp.float32)
    colsum = jnp.sum(assign, axis=0, keepdims=True)
    sn = assign / (colsum + _EPS)
    h_ref[0] = jnp.dot(sn, z, preferred_element_type=jnp.float32)


def _whole(shape):
    nd = len(shape)
    return pl.BlockSpec((1,) + shape[1:], lambda b: (b,) + (0,) * (nd - 1))


def _bcast(arr):
    return pl.BlockSpec(arr.shape, lambda b: (0,) * arr.ndim)


_PAR = pltpu.CompilerParams(dimension_semantics=("parallel",))


def _key_lanes(key, bsz):
    kd = jax.random.key_data(jax.random.split(key, bsz))
    k0 = jnp.broadcast_to(kd[:, 0:1], (bsz, 128)).reshape(bsz, 1, 128)
    k1 = jnp.broadcast_to(kd[:, 1:2], (bsz, 128)).reshape(bsz, 1, 128)
    return k0, k1


def _valid_rows(mask, k_max):
    bsz = mask.shape[0]
    k_list = jnp.ceil(_FILT * jnp.sum(mask, axis=1)).astype(jnp.int32)
    return (jax.lax.broadcasted_iota(jnp.int32, (bsz, k_max), 1)
            < k_list[:, None]).astype(jnp.float32)


def kernel(node_feat, mask_node, g0_w, g0_b, g1_w, g1_b, g2_w, g2_b,
           wb0, g3_w, g3_b, wb1, samp_key):
    bsz, n, _ = node_feat.shape
    hid = g0_w.shape[1]
    k0n = int(math.ceil(_FILT * n))
    k1n = int(math.ceil(_FILT * k0n))

    key = jax.random.key(samp_key)
    keys = jax.random.split(key, 2)
    ka0, ka1 = _key_lanes(keys[0], bsz)
    kb0, kb1 = _key_lanes(keys[1], bsz)

    adj, idx0, z0 = pl.pallas_call(
        _block0_body,
        out_shape=(jax.ShapeDtypeStruct((bsz, n, n), jnp.float32),
                   jax.ShapeDtypeStruct((bsz, k0n, 1), jnp.int32),
                   jax.ShapeDtypeStruct((bsz, n, hid), jnp.float32)),
        grid=(bsz,),
        in_specs=[_whole((bsz, n, node_feat.shape[2])),
                  _whole((bsz, n, 1)),
                  _whole((bsz, 1, 128)), _whole((bsz, 1, 128)),
                  _bcast(g0_w), _bcast(g0_b), _bcast(g1_w), _bcast(g1_b),
                  _bcast(g2_w), _bcast(g2_b), _bcast(wb0)],
        out_specs=(_whole((bsz, n, n)), _whole((bsz, k0n, 1)),
                   _whole((bsz, n, hid))),
        compiler_params=_PAR,
    )(node_feat, mask_node.reshape(bsz, n, 1), ka0, ka1, g0_w, g0_b,
      g1_w, g1_b, g2_w, g2_b, wb0)

    mask1 = _valid_rows(mask_node, k0n)

    nadj, idx1, z1 = pl.pallas_call(
        _pool_gcn_body,
        out_shape=(jax.ShapeDtypeStruct((bsz, k0n, k0n), jnp.float32),
                   jax.ShapeDtypeStruct((bsz, k1n, 1), jnp.int32),
                   jax.ShapeDtypeStruct((bsz, k0n, hid), jnp.float32)),
        grid=(bsz,),
        in_specs=[_whole((bsz, k0n, 1)), _whole((bsz, k0n, 1)),
                  _whole((bsz, 1, 128)), _whole((bsz, 1, 128)),
                  _whole((bsz, n, hid)), _whole((bsz, n, n)),
                  _bcast(g3_w), _bcast(g3_b), _bcast(wb1)],
        out_specs=(_whole((bsz, k0n, k0n)), _whole((bsz, k1n, 1)),
                   _whole((bsz, k0n, hid))),
        compiler_params=_PAR,
    )(idx0, mask1.reshape(bsz, k0n, 1), kb0, kb1, z0, adj, g3_w, g3_b, wb1)

    mask2 = _valid_rows(mask1, k1n)

    x_out = pl.pallas_call(
        _pool_final_body,
        out_shape=jax.ShapeDtypeStruct((bsz, k1n, hid), jnp.float32),
        grid=(bsz,),
        in_specs=[_whole((bsz, k1n, 1)), _whole((bsz, k1n, 1)),
                  _whole((bsz, k0n, hid)), _whole((bsz, k0n, k0n))],
        out_specs=_whole((bsz, k1n, hid)),
        compiler_params=_PAR,
    )(idx1, mask2.reshape(bsz, k1n, 1), z1, nadj)

    return x_out
